# baseline (speedup 1.0000x reference)
import functools

import numpy as np

import jax
import jax.numpy as jnp
from jax import lax
from jax.experimental import pallas as pl
from jax.experimental.pallas import tpu as pltpu
from jax.experimental.pallas import tpu_sc as plsc

_DG = 8
_R = 64
_NBUF = 4
_S_TC_FRAC_NUM = 5
_S_TC_FRAC_DEN = 8
_TC_BLK = 256


def _rotl(x, d):
    return ((x << np.uint32(d)) | (x >> np.uint32(32 - d))).astype(np.uint32)


def _threefry2x32(k1, k2, x0, x1):
    rot_a = [np.uint32(r) for r in (13, 15, 26, 6)]
    rot_b = [np.uint32(r) for r in (17, 29, 16, 24)]
    ks = [k1, k2, np.uint32(k1 ^ k2 ^ np.uint32(0x1BD11BDA))]
    x = [(x0 + ks[0]).astype(np.uint32), (x1 + ks[1]).astype(np.uint32)]

    def rounds(x, rots):
        for r in rots:
            x[0] = (x[0] + x[1]).astype(np.uint32)
            x[1] = (x[0] ^ _rotl(x[1], r)).astype(np.uint32)
        return x

    for i, (rots, ka, kb) in enumerate(
            [(rot_a, 1, 2), (rot_b, 2, 0), (rot_a, 0, 1),
             (rot_b, 1, 2), (rot_a, 2, 0)]):
        x = rounds(x, rots)
        x[0] = (x[0] + ks[ka]).astype(np.uint32)
        x[1] = (x[1] + ks[kb] + np.uint32(i + 1)).astype(np.uint32)
    return x[0], x[1]


@functools.lru_cache(maxsize=None)
def _gumbel_const(shape, dtype_name):
    n = int(np.prod(shape))
    k1, k2 = np.uint32(0), np.uint32(42)
    i64 = np.arange(n, dtype=np.uint64)
    c1 = (i64 >> np.uint64(32)).astype(np.uint32)
    c2 = (i64 & np.uint64(0xFFFFFFFF)).astype(np.uint32)
    b1, b2 = _threefry2x32(k1, k2, c1, c2)
    bits = (b1 ^ b2).reshape(shape)
    tiny = np.float32(np.finfo(np.float32).tiny)
    fb = (bits >> np.uint32(9)) | np.uint32(0x3F800000)
    floats = fb.view(np.float32) - np.float32(1.0)
    u = np.maximum(tiny, floats * (np.float32(1.0) - tiny) + tiny)
    return (-np.log(-np.log(u))).astype(np.dtype(dtype_name))


def _sc_reduce_body(s_base, s_len, x_hbm, psum_hbm, b0, b1, b2, b3, acc,
                    s0, s1, s2, s3):
    B, S, D = x_hbm.shape
    dslice = D // _DG
    nv = dslice // 16
    c = lax.axis_index("c")
    s = lax.axis_index("s")
    wid = s * 2 + c
    b = wid // _DG
    dg = wid % _DG
    d0 = dg * dslice
    nch = s_len // _R

    bufs = (b0, b1, b2, b3)
    sems = (s0, s1, s2, s3)

    def src(i):
        return x_hbm.at[b, pl.ds(s_base + i * _R, _R), pl.ds(d0, dslice)]

    for k in range(_NBUF):
        pltpu.async_copy(src(k), bufs[k], sems[k])

    def accum(buf, carry):
        def rb(k, carry):
            r = k * 4
            out = list(carry)
            for dr in range(4):
                for j in range(nv):
                    out[j] = out[j] + buf[r + dr, pl.ds(j * 16, 16)]
            return tuple(out)
        return lax.fori_loop(0, _R // 4, rb, carry)

    def pair_body(p, carry):
        i0 = p * _NBUF
        for k in range(_NBUF):
            i = i0 + k
            pltpu.make_async_copy(src(i), bufs[k], sems[k]).wait()
            carry = accum(bufs[k], carry)

            @pl.when(i + _NBUF < nch)
            def _():
                pltpu.async_copy(src(i + _NBUF), bufs[k], sems[k])
        return carry

    carry0 = tuple(jnp.zeros((16,), jnp.float32) for _ in range(nv))
    res = lax.fori_loop(0, nch // _NBUF, pair_body, carry0)
    for j in range(nv):
        acc[pl.ds(j * 16, 16)] = res[j]
    pltpu.sync_copy(acc, psum_hbm.at[b, pl.ds(d0, dslice)])


def _sc_partial_sums(x, s_base, s_len):
    B, S, D = x.shape
    dslice = D // _DG
    mesh = plsc.VectorSubcoreMesh(core_axis_name="c", subcore_axis_name="s")
    kfn = pl.kernel(
        functools.partial(_sc_reduce_body, s_base, s_len),
        mesh=mesh,
        out_type=jax.ShapeDtypeStruct((B, D), jnp.float32),
        scratch_types=[
            pltpu.VMEM((_R, dslice), jnp.float32),
            pltpu.VMEM((_R, dslice), jnp.float32),
            pltpu.VMEM((_R, dslice), jnp.float32),
            pltpu.VMEM((_R, dslice), jnp.float32),
            pltpu.VMEM((dslice,), jnp.float32),
            pltpu.SemaphoreType.DMA,
            pltpu.SemaphoreType.DMA,
            pltpu.SemaphoreType.DMA,
            pltpu.SemaphoreType.DMA,
        ],
    )
    return kfn(x)


def _tc_partial_kernel(x_ref, out_ref, acc_ref):
    i = pl.program_id(0)

    @pl.when(i == 0)
    def _init():
        acc_ref[...] = jnp.zeros_like(acc_ref)

    acc_ref[...] += jnp.sum(x_ref[...], axis=1)

    @pl.when(i == pl.num_programs(0) - 1)
    def _finish():
        out_ref[...] = acc_ref[...]


def _tc_partial_sum(x, s_len):
    B, S, D = x.shape
    grid = (s_len // _TC_BLK,)
    return pl.pallas_call(
        _tc_partial_kernel,
        grid=grid,
        in_specs=[pl.BlockSpec((B, _TC_BLK, D), lambda i: (0, i, 0))],
        out_specs=pl.BlockSpec((B, D), lambda i: (0, 0)),
        out_shape=jax.ShapeDtypeStruct((B, D), jnp.float32),
        scratch_shapes=[pltpu.VMEM((B, D), jnp.float32)],
        compiler_params=pltpu.CompilerParams(
            dimension_semantics=("arbitrary",),
        ),
    )(x)


def _tail_kernel(ptc_ref, psc_ref, w_ref, b_ref, g_ref, out_ref, *, inv_s):
    z = (ptc_ref[...] + psc_ref[...]) * inv_s
    logits = jax.lax.dot_general(
        z, w_ref[...], (((1,), (1,)), ((), ())),
        preferred_element_type=jnp.float32,
    )
    a = (logits + b_ref[...]) + g_ref[...]
    m = jnp.max(a, axis=-1, keepdims=True)
    e = jnp.exp(a - m)
    y = e / jnp.sum(e, axis=-1, keepdims=True)
    ymax = jnp.max(y, axis=-1, keepdims=True)
    iota = jax.lax.broadcasted_iota(jnp.int32, y.shape, 1)
    idx = jnp.min(jnp.where(y >= ymax, iota, y.shape[-1]), axis=-1,
                  keepdims=True)
    y_hard = (iota == idx).astype(y.dtype)
    out_ref[...] = (y_hard - y) + y


def kernel(x, W, b):
    B, S, D = x.shape
    E = W.shape[0]
    g = jnp.asarray(_gumbel_const((B, E), str(x.dtype)))
    b2 = b.reshape(1, E)
    s_tc = (S * _S_TC_FRAC_NUM // _S_TC_FRAC_DEN) // 512 * 512
    psc = _sc_partial_sums(x, s_tc, S - s_tc)
    ptc = _tc_partial_sum(x, s_tc)
    return pl.pallas_call(
        functools.partial(_tail_kernel, inv_s=1.0 / S),
        out_shape=jax.ShapeDtypeStruct((B, E), x.dtype),
    )(ptc, psc, W, b2, g)


# speedup vs baseline: 1.4007x; 1.4007x over previous
import functools

import numpy as np

import jax
import jax.numpy as jnp
from jax.experimental import pallas as pl
from jax.experimental.pallas import tpu as pltpu


def _rotl(x, d):
    return ((x << np.uint32(d)) | (x >> np.uint32(32 - d))).astype(np.uint32)


def _threefry2x32(k1, k2, x0, x1):
    rot_a = [np.uint32(r) for r in (13, 15, 26, 6)]
    rot_b = [np.uint32(r) for r in (17, 29, 16, 24)]
    ks = [k1, k2, np.uint32(k1 ^ k2 ^ np.uint32(0x1BD11BDA))]
    x = [(x0 + ks[0]).astype(np.uint32), (x1 + ks[1]).astype(np.uint32)]

    def rounds(x, rots):
        for r in rots:
            x[0] = (x[0] + x[1]).astype(np.uint32)
            x[1] = (x[0] ^ _rotl(x[1], r)).astype(np.uint32)
        return x

    for i, (rots, ka, kb) in enumerate(
            [(rot_a, 1, 2), (rot_b, 2, 0), (rot_a, 0, 1),
             (rot_b, 1, 2), (rot_a, 2, 0)]):
        x = rounds(x, rots)
        x[0] = (x[0] + ks[ka]).astype(np.uint32)
        x[1] = (x[1] + ks[kb] + np.uint32(i + 1)).astype(np.uint32)
    return x[0], x[1]


@functools.lru_cache(maxsize=None)
def _gumbel_const(shape, dtype_name):
    n = int(np.prod(shape))
    k1, k2 = np.uint32(0), np.uint32(42)
    i64 = np.arange(n, dtype=np.uint64)
    c1 = (i64 >> np.uint64(32)).astype(np.uint32)
    c2 = (i64 & np.uint64(0xFFFFFFFF)).astype(np.uint32)
    b1, b2 = _threefry2x32(k1, k2, c1, c2)
    bits = (b1 ^ b2).reshape(shape)
    tiny = np.float32(np.finfo(np.float32).tiny)
    fb = (bits >> np.uint32(9)) | np.uint32(0x3F800000)
    floats = fb.view(np.float32) - np.float32(1.0)
    u = np.maximum(tiny, floats * (np.float32(1.0) - tiny) + tiny)
    return (-np.log(-np.log(u))).astype(np.dtype(dtype_name))


def _router_kernel(x_ref, w_ref, b_ref, g_ref, out_ref, acc_ref):
    i = pl.program_id(0)

    @pl.when(i == 0)
    def _init():
        acc_ref[...] = jnp.zeros_like(acc_ref)

    acc_ref[...] += jnp.sum(x_ref[...], axis=1)

    @pl.when(i == pl.num_programs(0) - 1)
    def _finish():
        s_total = x_ref.shape[1] * pl.num_programs(0)
        z = acc_ref[...] * (1.0 / s_total)
        logits = jax.lax.dot_general(
            z, w_ref[...], (((1,), (1,)), ((), ())),
            preferred_element_type=jnp.float32,
        )
        a = (logits + b_ref[...]) + g_ref[...]
        m = jnp.max(a, axis=-1, keepdims=True)
        e = jnp.exp(a - m)
        y = e / jnp.sum(e, axis=-1, keepdims=True)
        ymax = jnp.max(y, axis=-1, keepdims=True)
        iota = jax.lax.broadcasted_iota(jnp.int32, y.shape, 1)
        idx = jnp.min(jnp.where(y >= ymax, iota, y.shape[-1]), axis=-1,
                      keepdims=True)
        y_hard = (iota == idx).astype(y.dtype)
        out_ref[...] = (y_hard - y) + y


def kernel(x, W, b):
    B, S, D = x.shape
    E = W.shape[0]
    g = jnp.asarray(_gumbel_const((B, E), str(x.dtype)))
    b2 = b.reshape(1, E)

    s_blk = 128
    grid = (S // s_blk,)

    return pl.pallas_call(
        _router_kernel,
        grid=grid,
        in_specs=[
            pl.BlockSpec((B, s_blk, D), lambda i: (0, i, 0)),
            pl.BlockSpec((E, D), lambda i: (0, 0)),
            pl.BlockSpec((1, E), lambda i: (0, 0)),
            pl.BlockSpec((B, E), lambda i: (0, 0)),
        ],
        out_specs=pl.BlockSpec((B, E), lambda i: (0, 0)),
        out_shape=jax.ShapeDtypeStruct((B, E), x.dtype),
        scratch_shapes=[pltpu.VMEM((B, D), jnp.float32)],
        compiler_params=pltpu.CompilerParams(
            dimension_semantics=("arbitrary",),
        ),
    )(x, W, b2, g)
